# Initial kernel scaffold; baseline (speedup 1.0000x reference)
import jax
import jax.numpy as jnp
from jax.experimental import pallas as pl


def kernel(x, table):
    raise NotImplementedError("write your pallas kernel here")


# speedup vs baseline: 3.4041x; 3.4041x over previous
import jax
import jax.numpy as jnp
from jax.experimental import pallas as pl


def _copy_block(t_ref, o_ref):
    o_ref[...] = t_ref[...]


def kernel(x, table):
    seq = x.shape[1]
    emb = table.shape[1]
    block = 512
    out = pl.pallas_call(
        _copy_block,
        grid=(seq // block,),
        in_specs=[pl.BlockSpec((block, emb), lambda i: (i, 0))],
        out_specs=pl.BlockSpec((block, emb), lambda i: (i, 0)),
        out_shape=jax.ShapeDtypeStruct((seq, emb), table.dtype),
    )(table)
    return out[None, :, :]


# speedup vs baseline: 3.6717x; 1.0786x over previous
import jax
import jax.numpy as jnp
from jax.experimental import pallas as pl


def _copy_block(t_ref, o_ref):
    o_ref[...] = t_ref[...]


def kernel(x, table):
    seq = x.shape[1]
    emb = table.shape[1]
    block = 1024
    out = pl.pallas_call(
        _copy_block,
        grid=(seq // block,),
        in_specs=[pl.BlockSpec((block, emb), lambda i: (i, 0))],
        out_specs=pl.BlockSpec((block, emb), lambda i: (i, 0)),
        out_shape=jax.ShapeDtypeStruct((seq, emb), table.dtype),
    )(table)
    return out[None, :, :]


# speedup vs baseline: 4.2267x; 1.1512x over previous
import jax
import jax.numpy as jnp
from jax.experimental import pallas as pl


def _copy_block(t_ref, o_ref):
    o_ref[...] = t_ref[...]


def kernel(x, table):
    seq = x.shape[1]
    emb = table.shape[1]
    block = 2048
    out = pl.pallas_call(
        _copy_block,
        grid=(seq // block,),
        in_specs=[pl.BlockSpec((block, emb), lambda i: (i, 0))],
        out_specs=pl.BlockSpec((block, emb), lambda i: (i, 0)),
        out_shape=jax.ShapeDtypeStruct((seq, emb), table.dtype),
    )(table)
    return out[None, :, :]
